# baseline (speedup 1.0000x reference)
import functools
import jax
import jax.numpy as jnp
from jax import lax
from jax.experimental import pallas as pl
from jax.experimental.pallas import tpu as pltpu
from jax.experimental.pallas import tpu_sc as plsc

_MODEL_DIM = 64
_BATCH = 4096
_SEQ = 200

_info = plsc.get_sparse_core_info()
_NC = _info.num_cores
_NS = _info.num_subcores
_NW = _NC * _NS
_BBLK = _BATCH // _NW

_mesh = plsc.VectorSubcoreMesh(core_axis_name="c", subcore_axis_name="s")


@functools.partial(
    pl.kernel,
    mesh=_mesh,
    out_type=jax.ShapeDtypeStruct((_SEQ, _MODEL_DIM, _BATCH), jnp.float32),
    scratch_types=[
        pltpu.VMEM((_SEQ, _BBLK), jnp.int32),
        pltpu.VMEM((_BBLK, _MODEL_DIM), jnp.float32),
        pltpu.VMEM((_BBLK, _MODEL_DIM), jnp.float32),
        pltpu.VMEM((_MODEL_DIM, _BBLK), jnp.float32),
        pltpu.VMEM((_MODEL_DIM, _BBLK), jnp.float32),
        pltpu.SemaphoreType.DMA,
        pltpu.SemaphoreType.DMA,
        pltpu.SemaphoreType.DMA,
        pltpu.SemaphoreType.DMA,
    ],
    compiler_params=pltpu.CompilerParams(
        use_tc_tiling_on_sc=False, needs_layout_passes=False
    ),
)
def _emb_lookup(
    ids_hbm, table_hbm, out_hbm,
    ids_v, rin0, rin1, rout0, rout1, gsem0, gsem1, osem0, osem1,
):
    rin = [rin0, rin1]
    rout = [rout0, rout1]
    gsem = [gsem0, gsem1]
    osem = [osem0, osem1]
    wid = lax.axis_index("s") * _NC + lax.axis_index("c")
    b0 = wid * _BBLK
    pltpu.sync_copy(ids_hbm.at[:, pl.ds(b0, _BBLK)], ids_v)
    pltpu.async_copy(table_hbm.at[ids_v.at[0]], rin0, gsem0)

    lane = lax.iota(jnp.int32, 16)
    rowvecs = [k * 16 + lane for k in range(_BBLK // 16)]

    def outer(g, carry):
        for p in range(2):
            s = 2 * g + p
            np_ = 1 - p
            pltpu.make_async_copy(
                table_hbm.at[ids_v.at[s]], rin[p], gsem[p]
            ).wait()

            @pl.when(jnp.logical_and(s >= 1, s + 1 < _SEQ))
            def _():
                pltpu.make_async_copy(
                    rout[np_],
                    out_hbm.at[s - 1, :, pl.ds(b0, _BBLK)],
                    osem[np_],
                ).wait()
                pltpu.async_copy(
                    table_hbm.at[ids_v.at[s + 1]], rin[np_], gsem[np_]
                )

            @pl.when(s == 0)
            def _():
                pltpu.async_copy(
                    table_hbm.at[ids_v.at[1]], rin[np_], gsem[np_]
                )

            def tbody(d, c):
                dvec = jnp.full((16,), d, jnp.int32)
                for k in range(_BBLK // 16):
                    v = plsc.load_gather(rin[p], [rowvecs[k], dvec])
                    rout[p][d, pl.ds(k * 16, 16)] = v * 8.0
                return c

            lax.fori_loop(0, _MODEL_DIM, tbody, 0, unroll=2)
            pltpu.async_copy(
                rout[p],
                out_hbm.at[s, :, pl.ds(b0, _BBLK)],
                osem[p],
            )
        return carry

    lax.fori_loop(0, _SEQ // 2, outer, 0)
    pltpu.make_async_copy(
        rout0, out_hbm.at[_SEQ - 2, :, pl.ds(b0, _BBLK)], osem0
    ).wait()
    pltpu.make_async_copy(
        rout1, out_hbm.at[_SEQ - 1, :, pl.ds(b0, _BBLK)], osem1
    ).wait()


def kernel(token_ids_batch, embeddings_table):
    ids_t = token_ids_batch.T.astype(jnp.int32)
    out = _emb_lookup(ids_t, embeddings_table)
    return jnp.transpose(out, (2, 0, 1))


# speedup vs baseline: 1.1835x; 1.1835x over previous
import functools
import jax
import jax.numpy as jnp
from jax import lax
from jax.experimental import pallas as pl
from jax.experimental.pallas import tpu as pltpu
from jax.experimental.pallas import tpu_sc as plsc

_MODEL_DIM = 64
_BATCH = 4096
_SEQ = 200

_info = plsc.get_sparse_core_info()
_NC = _info.num_cores
_NS = _info.num_subcores
_NW = _NC * _NS
_BBLK = _BATCH // _NW

_mesh = plsc.VectorSubcoreMesh(core_axis_name="c", subcore_axis_name="s")


@functools.partial(
    pl.kernel,
    mesh=_mesh,
    out_type=jax.ShapeDtypeStruct((_SEQ, _MODEL_DIM, _BATCH), jnp.float32),
    scratch_types=[
        pltpu.VMEM((_SEQ, _BBLK), jnp.int32),
        pltpu.VMEM((_BBLK, _MODEL_DIM), jnp.float32),
        pltpu.VMEM((_BBLK, _MODEL_DIM), jnp.float32),
        pltpu.VMEM((_MODEL_DIM, _BBLK), jnp.float32),
        pltpu.VMEM((_MODEL_DIM, _BBLK), jnp.float32),
        pltpu.SemaphoreType.DMA,
        pltpu.SemaphoreType.DMA,
        pltpu.SemaphoreType.DMA,
        pltpu.SemaphoreType.DMA,
    ],
    compiler_params=pltpu.CompilerParams(
        use_tc_tiling_on_sc=False, needs_layout_passes=False
    ),
)
def _emb_lookup(
    ids_hbm, table_hbm, out_hbm,
    ids_v, rin0, rin1, rout0, rout1, gsem0, gsem1, osem0, osem1,
):
    rin = [rin0, rin1]
    rout = [rout0, rout1]
    gsem = [gsem0, gsem1]
    osem = [osem0, osem1]
    wid = lax.axis_index("s") * _NC + lax.axis_index("c")
    b0 = wid * _BBLK
    pltpu.sync_copy(ids_hbm.at[:, pl.ds(b0, _BBLK)], ids_v)
    pltpu.async_copy(table_hbm.at[ids_v.at[0]], rin0, gsem0)

    lane = lax.iota(jnp.int32, 16)
    rowvecs = [k * 16 + lane for k in range(_BBLK // 16)]

    def outer(g, carry):
        for p in range(2):
            s = 2 * g + p
            np_ = 1 - p
            pltpu.make_async_copy(
                table_hbm.at[ids_v.at[s]], rin[p], gsem[p]
            ).wait()

            @pl.when(jnp.logical_and(s >= 1, s + 1 < _SEQ))
            def _():
                pltpu.make_async_copy(
                    rout[np_],
                    out_hbm.at[s - 1, :, pl.ds(b0, _BBLK)],
                    osem[np_],
                ).wait()
                pltpu.async_copy(
                    table_hbm.at[ids_v.at[s + 1]], rin[np_], gsem[np_]
                )

            @pl.when(s == 0)
            def _():
                pltpu.async_copy(
                    table_hbm.at[ids_v.at[1]], rin[np_], gsem[np_]
                )

            def tbody(d, c):
                dvec = jnp.full((16,), d, jnp.int32)
                vs = [
                    plsc.load_gather(rin[p], [rowvecs[k], dvec]) * 8.0
                    for k in range(_BBLK // 16)
                ]
                for k in range(_BBLK // 16):
                    rout[p][d, pl.ds(k * 16, 16)] = vs[k]
                return c

            lax.fori_loop(0, _MODEL_DIM, tbody, 0, unroll=2)
            pltpu.async_copy(
                rout[p],
                out_hbm.at[s, :, pl.ds(b0, _BBLK)],
                osem[p],
            )
        return carry

    lax.fori_loop(0, _SEQ // 2, outer, 0)
    pltpu.make_async_copy(
        rout0, out_hbm.at[_SEQ - 2, :, pl.ds(b0, _BBLK)], osem0
    ).wait()
    pltpu.make_async_copy(
        rout1, out_hbm.at[_SEQ - 1, :, pl.ds(b0, _BBLK)], osem1
    ).wait()


def kernel(token_ids_batch, embeddings_table):
    ids_t = token_ids_batch.T.astype(jnp.int32)
    out = _emb_lookup(ids_t, embeddings_table)
    return jnp.transpose(out, (2, 0, 1))


# speedup vs baseline: 1.1888x; 1.0044x over previous
import functools
import jax
import jax.numpy as jnp
from jax import lax
from jax.experimental import pallas as pl
from jax.experimental.pallas import tpu as pltpu
from jax.experimental.pallas import tpu_sc as plsc

_MODEL_DIM = 64
_BATCH = 4096
_SEQ = 200

_info = plsc.get_sparse_core_info()
_NC = _info.num_cores
_NS = _info.num_subcores
_SPC = _SEQ // _NC
_BBLK = _BATCH // _NS
_IBLK = 128
_NG = _BBLK // _IBLK

_mesh = plsc.VectorSubcoreMesh(core_axis_name="c", subcore_axis_name="s")


@functools.partial(
    pl.kernel,
    mesh=_mesh,
    out_type=jax.ShapeDtypeStruct((_SEQ, _MODEL_DIM, _BATCH), jnp.float32),
    scratch_types=[
        pltpu.VMEM((_SPC, _NG, _IBLK), jnp.int32),
        pltpu.VMEM((_BBLK, _MODEL_DIM), jnp.float32),
        pltpu.VMEM((_BBLK, _MODEL_DIM), jnp.float32),
        pltpu.VMEM((_MODEL_DIM, _BBLK), jnp.float32),
        pltpu.VMEM((_MODEL_DIM, _BBLK), jnp.float32),
        pltpu.SemaphoreType.DMA,
        pltpu.SemaphoreType.DMA,
        pltpu.SemaphoreType.DMA,
        pltpu.SemaphoreType.DMA,
    ],
    compiler_params=pltpu.CompilerParams(
        use_tc_tiling_on_sc=False, needs_layout_passes=False
    ),
)
def _emb_lookup(
    ids_hbm, table_hbm, out_hbm,
    ids_v, rin0, rin1, rout0, rout1, gsem0, gsem1, osem0, osem1,
):
    rin = [rin0, rin1]
    rout = [rout0, rout1]
    gsem = [gsem0, gsem1]
    osem = [osem0, osem1]
    cid = lax.axis_index("c")
    sid = lax.axis_index("s")
    s0 = cid * _SPC
    b0 = sid * _BBLK
    for j in range(_NG):
        pltpu.sync_copy(
            ids_hbm.at[pl.ds(s0, _SPC), pl.ds(b0 + j * _IBLK, _IBLK)],
            ids_v.at[:, j, :],
        )

    def gather_pos(li, buf, sem):
        for j in range(_NG):
            pltpu.async_copy(
                table_hbm.at[ids_v.at[li, j]],
                buf.at[pl.ds(j * _IBLK, _IBLK)],
                sem,
            )

    def wait_pos(li, buf, sem):
        for j in range(_NG):
            pltpu.make_async_copy(
                table_hbm.at[ids_v.at[li, j]],
                buf.at[pl.ds(j * _IBLK, _IBLK)],
                sem,
            ).wait()

    gather_pos(0, rin0, gsem0)

    lane = lax.iota(jnp.int32, 16)
    rowvecs = [k * 16 + lane for k in range(_BBLK // 16)]

    def outer(g, carry):
        for p in range(2):
            li = 2 * g + p
            np_ = 1 - p
            wait_pos(li, rin[p], gsem[p])

            @pl.when(jnp.logical_and(li >= 1, li + 1 < _SPC))
            def _():
                pltpu.make_async_copy(
                    rout[np_],
                    out_hbm.at[s0 + li - 1, :, pl.ds(b0, _BBLK)],
                    osem[np_],
                ).wait()
                gather_pos(li + 1, rin[np_], gsem[np_])

            @pl.when(li == 0)
            def _():
                gather_pos(1, rin[np_], gsem[np_])

            def tbody(d, c):
                dvec = jnp.full((16,), d, jnp.int32)
                vs = [
                    plsc.load_gather(rin[p], [rowvecs[k], dvec]) * 8.0
                    for k in range(_BBLK // 16)
                ]
                for k in range(_BBLK // 16):
                    rout[p][d, pl.ds(k * 16, 16)] = vs[k]
                return c

            lax.fori_loop(0, _MODEL_DIM, tbody, 0, unroll=2)
            pltpu.async_copy(
                rout[p],
                out_hbm.at[s0 + li, :, pl.ds(b0, _BBLK)],
                osem[p],
            )
        return carry

    lax.fori_loop(0, _SPC // 2, outer, 0)
    pltpu.make_async_copy(
        rout0, out_hbm.at[s0 + _SPC - 2, :, pl.ds(b0, _BBLK)], osem0
    ).wait()
    pltpu.make_async_copy(
        rout1, out_hbm.at[s0 + _SPC - 1, :, pl.ds(b0, _BBLK)], osem1
    ).wait()


def kernel(token_ids_batch, embeddings_table):
    ids_t = token_ids_batch.T.astype(jnp.int32)
    out = _emb_lookup(ids_t, embeddings_table)
    return jnp.transpose(out, (2, 0, 1))


# speedup vs baseline: 1.8995x; 1.5978x over previous
import functools
import jax
import jax.numpy as jnp
from jax import lax
from jax.experimental import pallas as pl
from jax.experimental.pallas import tpu as pltpu
from jax.experimental.pallas import tpu_sc as plsc

_MODEL_DIM = 64
_BATCH = 4096
_SEQ = 200

_info = plsc.get_sparse_core_info()
_NC = _info.num_cores
_NS = _info.num_subcores
_SPC = _SEQ // _NC
_BBLK = _BATCH // _NS
_IBLK = 128
_NG = _BBLK // _IBLK
_TPAD = _BBLK + 1

_mesh = plsc.VectorSubcoreMesh(core_axis_name="c", subcore_axis_name="s")


@functools.partial(
    pl.kernel,
    mesh=_mesh,
    out_type=jax.ShapeDtypeStruct((_SEQ, _MODEL_DIM, _BATCH), jnp.float32),
    scratch_types=[
        pltpu.VMEM((_SPC, _NG, _IBLK), jnp.int32),
        pltpu.VMEM((_BBLK, _MODEL_DIM), jnp.float32),
        pltpu.VMEM((_BBLK, _MODEL_DIM), jnp.float32),
        pltpu.VMEM((_MODEL_DIM, _TPAD), jnp.float32),
        pltpu.VMEM_SHARED((_MODEL_DIM, _BATCH), jnp.float32),
        pltpu.VMEM_SHARED((_MODEL_DIM, _BATCH), jnp.float32),
        pltpu.SemaphoreType.DMA,
        pltpu.SemaphoreType.DMA,
        pltpu.SemaphoreType.DMA,
        pltpu.SemaphoreType.DMA,
    ],
    compiler_params=pltpu.CompilerParams(
        use_tc_tiling_on_sc=False, needs_layout_passes=False
    ),
)
def _emb_lookup(
    ids_hbm, table_hbm, out_hbm,
    ids_v, rin0, rin1, rout, blk0, blk1, gsem0, gsem1, osem0, osem1,
):
    rin = [rin0, rin1]
    blk = [blk0, blk1]
    gsem = [gsem0, gsem1]
    osem = [osem0, osem1]
    cid = lax.axis_index("c")
    sid = lax.axis_index("s")
    s0 = cid * _SPC
    b0 = sid * _BBLK
    for j in range(_NG):
        pltpu.sync_copy(
            ids_hbm.at[pl.ds(s0, _SPC), pl.ds(b0 + j * _IBLK, _IBLK)],
            ids_v.at[:, j, :],
        )

    def gather_pos(li, buf, sem):
        for j in range(_NG):
            pltpu.async_copy(
                table_hbm.at[ids_v.at[li, j]],
                buf.at[pl.ds(j * _IBLK, _IBLK)],
                sem,
            )

    def wait_pos(li, buf, sem):
        for j in range(_NG):
            pltpu.make_async_copy(
                table_hbm.at[ids_v.at[li, j]],
                buf.at[pl.ds(j * _IBLK, _IBLK)],
                sem,
            ).wait()

    gather_pos(0, rin0, gsem0)

    lane = lax.iota(jnp.int32, 16)
    dvecs = [k * 16 + lane for k in range(_MODEL_DIM // 16)]

    def outer(g, carry):
        for p in range(2):
            li = 2 * g + p
            wait_pos(li, rin[p], gsem[p])

            @pl.when(li + 1 < _SPC)
            def _():
                gather_pos(li + 1, rin[1 - p], gsem[1 - p])

            def tbody(r, c):
                rvec = jnp.full((16,), r, jnp.int32)
                vs = [
                    rin[p][r, pl.ds(k * 16, 16)] * 8.0
                    for k in range(_MODEL_DIM // 16)
                ]
                for k in range(_MODEL_DIM // 16):
                    plsc.store_scatter(rout, [dvecs[k], rvec], vs[k])
                return c

            lax.fori_loop(0, _BBLK, tbody, 0, unroll=2)

            @pl.when(jnp.logical_and(sid == 0, li >= 2))
            def _():
                pltpu.make_async_copy(
                    blk[p], out_hbm.at[s0 + li - 2], osem[p]
                ).wait()

            plsc.subcore_barrier()
            pltpu.sync_copy(
                rout.at[:, pl.ds(0, _BBLK)],
                blk[p].at[:, pl.ds(b0, _BBLK)],
            )
            plsc.subcore_barrier()

            @pl.when(sid == 0)
            def _():
                pltpu.async_copy(blk[p], out_hbm.at[s0 + li], osem[p])
        return carry

    lax.fori_loop(0, _SPC // 2, outer, 0)

    @pl.when(sid == 0)
    def _():
        pltpu.make_async_copy(
            blk0, out_hbm.at[s0 + _SPC - 2], osem0
        ).wait()
        pltpu.make_async_copy(
            blk1, out_hbm.at[s0 + _SPC - 1], osem1
        ).wait()


def kernel(token_ids_batch, embeddings_table):
    ids_t = token_ids_batch.T.astype(jnp.int32)
    out = _emb_lookup(ids_t, embeddings_table)
    return jnp.transpose(out, (2, 0, 1))


# speedup vs baseline: 2.3774x; 1.2516x over previous
import functools
import jax
import jax.numpy as jnp
from jax import lax
from jax.experimental import pallas as pl
from jax.experimental.pallas import tpu as pltpu
from jax.experimental.pallas import tpu_sc as plsc

_MODEL_DIM = 64
_BATCH = 4096
_SEQ = 200
_VOCAB = 1000000

_info = plsc.get_sparse_core_info()
_NC = _info.num_cores
_NS = _info.num_subcores
_SPC = _SEQ // _NC
_BBLK = _BATCH // _NS
_IBLK = 128
_NG = _BBLK // _IBLK
_TPAD = _BBLK + 1
_DT = _MODEL_DIM // 8
_BT = _BATCH // 128

_mesh = plsc.VectorSubcoreMesh(core_axis_name="c", subcore_axis_name="s")


@functools.partial(
    pl.kernel,
    mesh=_mesh,
    out_type=jax.ShapeDtypeStruct((_SEQ, _DT, _BT, 8, 128), jnp.float32),
    scratch_types=[
        pltpu.VMEM((_SPC, _NG, _IBLK), jnp.int32),
        pltpu.VMEM((_BBLK, _MODEL_DIM), jnp.float32),
        pltpu.VMEM((_BBLK, _MODEL_DIM), jnp.float32),
        pltpu.VMEM((_MODEL_DIM, _TPAD), jnp.float32),
        pltpu.VMEM((_MODEL_DIM, _TPAD), jnp.float32),
        pltpu.SemaphoreType.DMA,
        pltpu.SemaphoreType.DMA,
        pltpu.SemaphoreType.DMA,
        pltpu.SemaphoreType.DMA,
    ],
    compiler_params=pltpu.CompilerParams(
        use_tc_tiling_on_sc=False, needs_layout_passes=False
    ),
)
def _emb_lookup(
    ids_hbm, table_hbm, out_hbm,
    ids_v, rin0, rin1, rout0, rout1, gsem0, gsem1, osem0, osem1,
):
    rin = [rin0, rin1]
    rout = [rout0, rout1]
    gsem = [gsem0, gsem1]
    osem = [osem0, osem1]
    cid = lax.axis_index("c")
    sid = lax.axis_index("s")
    s0 = cid * _SPC
    b0 = sid * _BBLK
    for j in range(_NG):
        pltpu.sync_copy(
            ids_hbm.at[pl.ds(s0, _SPC), pl.ds(b0 + j * _IBLK, _IBLK)],
            ids_v.at[:, j, :],
        )

    def gather_pos(li, buf, sem):
        for j in range(_NG):
            pltpu.async_copy(
                table_hbm.at[ids_v.at[li, j]],
                buf.at[pl.ds(j * _IBLK, _IBLK)],
                sem,
            )

    def wait_pos(li, buf, sem):
        for j in range(_NG):
            pltpu.make_async_copy(
                table_hbm.at[ids_v.at[li, j]],
                buf.at[pl.ds(j * _IBLK, _IBLK)],
                sem,
            ).wait()

    def out_chunks(li, p):
        for dt in range(_DT):
            for bt in range(_NG):
                yield (
                    rout[p].at[pl.ds(8 * dt, 8), pl.ds(128 * bt, 128)],
                    out_hbm.at[s0 + li, dt, _NG * sid + bt],
                )

    gather_pos(0, rin0, gsem0)

    lane = lax.iota(jnp.int32, 16)
    dvecs = [k * 16 + lane for k in range(_MODEL_DIM // 16)]

    def outer(g, carry):
        for p in range(2):
            li = 2 * g + p
            wait_pos(li, rin[p], gsem[p])

            @pl.when(li + 1 < _SPC)
            def _():
                gather_pos(li + 1, rin[1 - p], gsem[1 - p])

            @pl.when(li >= 2)
            def _():
                for src, dst in out_chunks(li - 2, p):
                    pltpu.make_async_copy(src, dst, osem[p]).wait()

            def tbody(r, c):
                rvec = jnp.full((16,), r, jnp.int32)
                vs = [
                    rin[p][r, pl.ds(k * 16, 16)] * 8.0
                    for k in range(_MODEL_DIM // 16)
                ]
                for k in range(_MODEL_DIM // 16):
                    plsc.store_scatter(rout[p], [dvecs[k], rvec], vs[k])
                return c

            lax.fori_loop(0, _BBLK, tbody, 0, unroll=2)

            for src, dst in out_chunks(li, p):
                pltpu.async_copy(src, dst, osem[p])
        return carry

    lax.fori_loop(0, _SPC // 2, outer, 0)

    for p, li in ((0, _SPC - 2), (1, _SPC - 1)):
        for src, dst in out_chunks(li, p):
            pltpu.make_async_copy(src, dst, osem[p]).wait()


def kernel(token_ids_batch, embeddings_table):
    ids_t = token_ids_batch.T.astype(jnp.int32)
    tbl_flat = lax.optimization_barrier(
        jnp.reshape(embeddings_table, (_MODEL_DIM * _VOCAB,))
    )
    tbl_lin = jnp.reshape(tbl_flat, (_VOCAB, _MODEL_DIM))
    out5 = _emb_lookup(ids_t, tbl_lin)
    out = jnp.transpose(out5, (2, 4, 0, 1, 3))
    return jnp.reshape(out, (_BATCH, _SEQ, _MODEL_DIM))
